# baseline (speedup 1.0000x reference)
import functools

import jax
import jax.numpy as jnp
from jax import lax
from jax.experimental import pallas as pl
from jax.experimental.pallas import tpu as pltpu
from jax.experimental.pallas import tpu_sc as plsc

N_NODES = 10000
E = 320000
D = 128
HALF = 64
NSUB = 16
C = 80
EDGES_PER_TILE = E // NSUB
CHUNKS = EDGES_PER_TILE // C
NODES_PER_TILE = N_NODES // NSUB
FCH = 25
FCHN = NODES_PER_TILE // FCH
L = 16


def _body(x_hbm, idx_hbm, beta_hbm, out_hbm,
          xb0, xb1, eb0, eb1, ib0, ib1, fbuf, obuf, bbuf, acc,
          sin0, sin1, ss0, ss1):
    c = lax.axis_index("c")
    s = lax.axis_index("s")
    xb, eb, ib = (xb0, xb1), (eb0, eb1), (ib0, ib1)
    sin, ss = (sin0, sin1), (ss0, ss1)

    pltpu.sync_copy(beta_hbm, bbuf)
    betav = bbuf[...]
    blog2e = betav * jnp.float32(1.4426950408889634)

    def _zrow(i, carry):
        for j in range(2):
            for k in range(HALF // L):
                fbuf[i, j, pl.ds(k * L, L)] = jnp.zeros((L,), jnp.float32)
        return carry
    lax.fori_loop(0, FCH, _zrow, 0)
    for k in range(FCHN):
        pltpu.sync_copy(fbuf, acc.at[pl.ds(s * NODES_PER_TILE + k * FCH, FCH)])
    plsc.subcore_barrier()

    def _start_in(g, b):
        e0 = pl.multiple_of(s * EDGES_PER_TILE + g * C, 8)
        pltpu.async_copy(x_hbm.at[pl.ds(e0, C), pl.ds(c * HALF, HALF)],
                         xb[b], sin[b])
        pltpu.async_copy(idx_hbm.at[pl.ds(e0, C)], ib[b], sin[b])

    def _wait_in(b):
        pltpu.make_async_copy(x_hbm.at[pl.ds(0, C), pl.ds(0, HALF)],
                              xb[b], sin[b]).wait()
        pltpu.make_async_copy(idx_hbm.at[pl.ds(0, C)], ib[b], sin[b]).wait()

    def _compute(b):
        @plsc.parallel_loop(0, C, unroll=8)
        def _vrow(i):
            for k in range(HALF // L):
                v = xb[b][i, pl.ds(k * L, L)]
                ez = jnp.exp(betav * v)
                eb[b][i, 0, pl.ds(k * L, L)] = ez * v
                eb[b][i, 1, pl.ds(k * L, L)] = ez

    def _start_scat(b):
        pltpu.async_copy(eb[b], acc.at[ib[b]], ss[b], add=True)

    def _wait_scat(b):
        pltpu.make_async_copy(eb[b], acc.at[ib[b]], ss[b]).wait()

    _start_in(0, 0)
    _start_in(1, 1)
    for b in range(2):
        _wait_in(b)
        _compute(b)
        _start_scat(b)
        _start_in(2 + b, b)

    def _pair(g2, carry):
        for b in range(2):
            g = 2 * g2 + b
            _wait_in(b)
            _wait_scat(b)
            _compute(b)
            _start_scat(b)

            @pl.when(g + 2 < CHUNKS)
            def _():
                _start_in(g + 2, b)
        return carry
    lax.fori_loop(1, CHUNKS // 2, _pair, 0)
    _wait_scat(0)
    _wait_scat(1)
    plsc.subcore_barrier()

    def _fin(k, carry):
        n0 = s * NODES_PER_TILE + k * FCH
        pltpu.sync_copy(acc.at[pl.ds(n0, FCH)], fbuf)

        def _frow(i, carry2):
            for kk in range(HALF // L):
                num = fbuf[i, 0, pl.ds(kk * L, L)]
                den = fbuf[i, 1, pl.ds(kk * L, L)]
                obuf[i, pl.ds(kk * L, L)] = jnp.where(
                    den > 0.0, num / den, jnp.zeros((L,), jnp.float32))
            return carry2
        lax.fori_loop(0, FCH, _frow, 0)

        pltpu.sync_copy(obuf, out_hbm.at[pl.ds(n0, FCH), pl.ds(c * HALF, HALF)])
        return carry
    lax.fori_loop(0, FCHN, _fin, 0)


def kernel(x, idx, dim, dim_size, beta):
    del dim, dim_size
    bvec = jnp.broadcast_to(jnp.asarray(beta, jnp.float32), (L,))
    mesh = plsc.VectorSubcoreMesh(core_axis_name="c", subcore_axis_name="s")
    f = functools.partial(
        pl.kernel,
        mesh=mesh,
        compiler_params=pltpu.CompilerParams(use_tc_tiling_on_sc=False),
        out_type=jax.ShapeDtypeStruct((N_NODES, D), jnp.float32),
        scratch_types=[
            pltpu.VMEM((C, HALF), jnp.float32),
            pltpu.VMEM((C, HALF), jnp.float32),
            pltpu.VMEM((C, 2, HALF), jnp.float32),
            pltpu.VMEM((C, 2, HALF), jnp.float32),
            pltpu.VMEM((C,), jnp.int32),
            pltpu.VMEM((C,), jnp.int32),
            pltpu.VMEM((FCH, 2, HALF), jnp.float32),
            pltpu.VMEM((FCH, HALF), jnp.float32),
            pltpu.VMEM((L,), jnp.float32),
            pltpu.VMEM_SHARED((N_NODES, 2, HALF), jnp.float32),
            pltpu.SemaphoreType.DMA,
            pltpu.SemaphoreType.DMA,
            pltpu.SemaphoreType.DMA,
            pltpu.SemaphoreType.DMA,
        ],
    )(_body)
    return f(x, idx, bvec)


# speedup vs baseline: 1.0326x; 1.0326x over previous
import functools

import jax
import jax.numpy as jnp
from jax import lax
from jax.experimental import pallas as pl
from jax.experimental.pallas import tpu as pltpu
from jax.experimental.pallas import tpu_sc as plsc

N_NODES = 10000
E = 320000
D = 128
HALF = 64
NSUB = 16
C = 80
EDGES_PER_TILE = E // NSUB
CHUNKS = EDGES_PER_TILE // C
NODES_PER_TILE = N_NODES // NSUB
FCH = 25
FCHN = NODES_PER_TILE // FCH
L = 16


NEB = 3
NIB = 6


def _body(x_hbm, idx_hbm, beta_hbm, out_hbm,
          xb0, xb1, eb0, eb1, eb2, ib0, ib1, ib2, ib3, ib4, ib5,
          fbuf, obuf, bbuf, acc, sin0, sin1, ss):
    c = lax.axis_index("c")
    s = lax.axis_index("s")
    xb = (xb0, xb1)
    eb = (eb0, eb1, eb2)
    ib = (ib0, ib1, ib2, ib3, ib4, ib5)
    sin = (sin0, sin1)

    pltpu.sync_copy(beta_hbm, bbuf)
    betav = bbuf[...]
    blog2e = betav * jnp.float32(1.4426950408889634)

    def _zrow(i, carry):
        for j in range(2):
            for k in range(HALF // L):
                fbuf[i, j, pl.ds(k * L, L)] = jnp.zeros((L,), jnp.float32)
        return carry
    lax.fori_loop(0, FCH, _zrow, 0)
    for k in range(FCHN):
        pltpu.sync_copy(fbuf, acc.at[pl.ds(s * NODES_PER_TILE + k * FCH, FCH)])
    plsc.subcore_barrier()

    def _start_in(g, bx, bi):
        e0 = pl.multiple_of(s * EDGES_PER_TILE + g * C, 8)
        pltpu.async_copy(x_hbm.at[pl.ds(e0, C), pl.ds(c * HALF, HALF)],
                         xb[bx], sin[bx])
        pltpu.async_copy(idx_hbm.at[pl.ds(e0, C)], ib[bi], sin[bx])

    def _wait_in(bx, bi):
        pltpu.make_async_copy(x_hbm.at[pl.ds(0, C), pl.ds(0, HALF)],
                              xb[bx], sin[bx]).wait()
        pltpu.make_async_copy(idx_hbm.at[pl.ds(0, C)], ib[bi], sin[bx]).wait()

    def _compute(bx, be):
        @plsc.parallel_loop(0, C, unroll=4)
        def _vrow(i):
            for k in range(HALF // L):
                v = xb[bx][i, pl.ds(k * L, L)]
                ez = jnp.exp(betav * v)
                eb[be][i, 0, pl.ds(k * L, L)] = ez * v
                eb[be][i, 1, pl.ds(k * L, L)] = ez

    def _start_scat(be, bi):
        pltpu.async_copy(eb[be], acc.at[ib[bi]], ss, add=True)

    def _drain_scat_one():
        pltpu.make_async_copy(eb[0], acc.at[ib[0]], ss).wait()

    def _chunk(g, u, drain):
        bx, be, bi = u % 2, u % NEB, u % NIB
        _wait_in(bx, bi)
        if drain:
            _drain_scat_one()
        _compute(bx, be)
        _start_scat(be, bi)

        @pl.when(g + 2 < CHUNKS)
        def _():
            _start_in(g + 2, bx, (bi + 2) % NIB)

    _start_in(0, 0, 0)
    _start_in(1, 1, 1)
    for u in range(NIB):
        _chunk(u, u, u >= NEB)

    def _six(i, carry):
        for u in range(NIB):
            _chunk(NIB * i + u, u, True)
        return carry
    lax.fori_loop(1, CHUNKS // NIB, _six, 0)
    for u in range(CHUNKS % NIB):
        _chunk(CHUNKS - (CHUNKS % NIB) + u, u, True)
    for _ in range(NEB):
        _drain_scat_one()
    plsc.subcore_barrier()

    def _fin(k, carry):
        n0 = s * NODES_PER_TILE + k * FCH
        pltpu.sync_copy(acc.at[pl.ds(n0, FCH)], fbuf)

        def _frow(i, carry2):
            for kk in range(HALF // L):
                num = fbuf[i, 0, pl.ds(kk * L, L)]
                den = fbuf[i, 1, pl.ds(kk * L, L)]
                obuf[i, pl.ds(kk * L, L)] = jnp.where(
                    den > 0.0, num / den, jnp.zeros((L,), jnp.float32))
            return carry2
        lax.fori_loop(0, FCH, _frow, 0)

        pltpu.sync_copy(obuf, out_hbm.at[pl.ds(n0, FCH), pl.ds(c * HALF, HALF)])
        return carry
    lax.fori_loop(0, FCHN, _fin, 0)


def kernel(x, idx, dim, dim_size, beta):
    del dim, dim_size
    bvec = jnp.broadcast_to(jnp.asarray(beta, jnp.float32), (L,))
    mesh = plsc.VectorSubcoreMesh(core_axis_name="c", subcore_axis_name="s")
    f = functools.partial(
        pl.kernel,
        mesh=mesh,
        compiler_params=pltpu.CompilerParams(use_tc_tiling_on_sc=False),
        out_type=jax.ShapeDtypeStruct((N_NODES, D), jnp.float32),
        scratch_types=(
            [pltpu.VMEM((C, HALF), jnp.float32)] * 2
            + [pltpu.VMEM((C, 2, HALF), jnp.float32)] * NEB
            + [pltpu.VMEM((C,), jnp.int32)] * NIB
            + [
                pltpu.VMEM((FCH, 2, HALF), jnp.float32),
                pltpu.VMEM((FCH, HALF), jnp.float32),
                pltpu.VMEM((L,), jnp.float32),
                pltpu.VMEM_SHARED((N_NODES, 2, HALF), jnp.float32),
                pltpu.SemaphoreType.DMA,
                pltpu.SemaphoreType.DMA,
                pltpu.SemaphoreType.DMA,
            ]
        ),
    )(_body)
    return f(x, idx, bvec)


# speedup vs baseline: 1.1764x; 1.1392x over previous
import functools

import jax
import jax.numpy as jnp
from jax import lax
from jax.experimental import pallas as pl
from jax.experimental.pallas import tpu as pltpu
from jax.experimental.pallas import tpu_sc as plsc

N_NODES = 10000
E = 320000
D = 128
HALF = 64
NSUB = 16
C = 80
EDGES_PER_TILE = E // NSUB
CHUNKS = EDGES_PER_TILE // C
NODES_PER_TILE = N_NODES // NSUB
FCH = 25
FCHN = NODES_PER_TILE // FCH
L = 16
NIB = 4


def _body(x_hbm, idx_hbm, beta_hbm, out_hbm,
          xb0, xb1, eb0, eb1, ib0, ib1, ib2, ib3,
          fb0, fb1, ob0, ob1, bbuf, acc, sin0, sin1, ss, sow0, sow1):
    c = lax.axis_index("c")
    s = lax.axis_index("s")
    xb = (xb0, xb1)
    eb = (eb0, eb1)
    ib = (ib0, ib1, ib2, ib3)
    fb = (fb0, fb1)
    ob = (ob0, ob1)
    sin = (sin0, sin1)
    sow = (sow0, sow1)

    pltpu.sync_copy(beta_hbm, bbuf)
    betav = bbuf[...]

    def _zrow(i, carry):
        for j in range(2):
            for k in range(HALF // L):
                fb0[i, j, pl.ds(k * L, L)] = jnp.zeros((L,), jnp.float32)
        return carry
    lax.fori_loop(0, FCH, _zrow, 0)
    for k in range(FCHN):
        pltpu.async_copy(fb0, acc.at[pl.ds(s * NODES_PER_TILE + k * FCH, FCH)],
                         ss)
    for k in range(FCHN):
        pltpu.make_async_copy(fb0, acc.at[pl.ds(0, FCH)], ss).wait()
    plsc.subcore_barrier()

    def _start_in(g, bx, bi):
        e0 = pl.multiple_of(s * EDGES_PER_TILE + g * C, 8)
        pltpu.async_copy(x_hbm.at[pl.ds(e0, C), pl.ds(c * HALF, HALF)],
                         xb[bx], sin[bx])
        pltpu.async_copy(idx_hbm.at[pl.ds(e0, C)], ib[bi], sin[bx])

    def _wait_in(bx, bi):
        pltpu.make_async_copy(x_hbm.at[pl.ds(0, C), pl.ds(0, HALF)],
                              xb[bx], sin[bx]).wait()
        pltpu.make_async_copy(idx_hbm.at[pl.ds(0, C)], ib[bi], sin[bx]).wait()

    def _compute(bx, be):
        @plsc.parallel_loop(0, C, unroll=4)
        def _vrow(i):
            for k in range(HALF // L):
                v = xb[bx][i, pl.ds(k * L, L)]
                ez = jnp.exp(betav * v)
                eb[be][i, 0, pl.ds(k * L, L)] = ez * v
                eb[be][i, 1, pl.ds(k * L, L)] = ez

    def _drain_scat_one():
        pltpu.make_async_copy(eb[0], acc.at[ib[0]], ss).wait()

    def _chunk(g, u, drain):
        bx, be, bi = u % 2, u % 2, u % NIB
        _wait_in(bx, bi)
        if drain:
            _drain_scat_one()
        _compute(bx, be)
        pltpu.async_copy(eb[be], acc.at[ib[bi]], ss, add=True)

        @pl.when(g + 2 < CHUNKS)
        def _():
            _start_in(g + 2, bx, (bi + 2) % NIB)

    _start_in(0, 0, 0)
    _start_in(1, 1, 1)
    _chunk(0, 0, False)
    _chunk(1, 1, False)

    def _quad(i, carry):
        for u in range(NIB):
            _chunk(2 + NIB * i + u, 2 + u, True)
        return carry
    lax.fori_loop(0, (CHUNKS - 2) // NIB, _quad, 0)
    _drain_scat_one()
    _drain_scat_one()
    plsc.subcore_barrier()

    def _start_f(k, b):
        n0 = s * NODES_PER_TILE + k * FCH
        pltpu.async_copy(acc.at[pl.ds(n0, FCH)], fb[b], sin[b])

    def _fchunk(k, b, drain_out):
        pltpu.make_async_copy(acc.at[pl.ds(0, FCH)], fb[b], sin[b]).wait()
        if drain_out:
            pltpu.make_async_copy(
                ob[b], out_hbm.at[pl.ds(0, FCH), pl.ds(0, HALF)], sow[b]).wait()

        @plsc.parallel_loop(0, FCH, unroll=5)
        def _frow(i):
            for kk in range(HALF // L):
                num = fb[b][i, 0, pl.ds(kk * L, L)]
                den = fb[b][i, 1, pl.ds(kk * L, L)]
                ob[b][i, pl.ds(kk * L, L)] = jnp.where(
                    den > 0.0, num / den, jnp.zeros((L,), jnp.float32))

        n0 = s * NODES_PER_TILE + k * FCH
        pltpu.async_copy(ob[b], out_hbm.at[pl.ds(n0, FCH), pl.ds(c * HALF, HALF)],
                         sow[b])

        @pl.when(k + 2 < FCHN)
        def _():
            _start_f(k + 2, b)

    _start_f(0, 0)
    _start_f(1, 1)
    _fchunk(0, 0, False)
    _fchunk(1, 1, False)

    def _fpair(j, carry):
        for u in range(2):
            _fchunk(2 + 2 * j + u, u, True)
        return carry
    lax.fori_loop(0, (FCHN - 2) // 2, _fpair, 0)
    _fchunk(FCHN - 1, 0, True)
    pltpu.make_async_copy(ob[0], out_hbm.at[pl.ds(0, FCH), pl.ds(0, HALF)],
                          sow[0]).wait()
    pltpu.make_async_copy(ob[1], out_hbm.at[pl.ds(0, FCH), pl.ds(0, HALF)],
                          sow[1]).wait()


def kernel(x, idx, dim, dim_size, beta):
    del dim, dim_size
    bvec = jnp.broadcast_to(jnp.asarray(beta, jnp.float32), (L,))
    mesh = plsc.VectorSubcoreMesh(core_axis_name="c", subcore_axis_name="s")
    f = functools.partial(
        pl.kernel,
        mesh=mesh,
        compiler_params=pltpu.CompilerParams(use_tc_tiling_on_sc=False),
        out_type=jax.ShapeDtypeStruct((N_NODES, D), jnp.float32),
        scratch_types=(
            [pltpu.VMEM((C, HALF), jnp.float32)] * 2
            + [pltpu.VMEM((C, 2, HALF), jnp.float32)] * 2
            + [pltpu.VMEM((C,), jnp.int32)] * NIB
            + [pltpu.VMEM((FCH, 2, HALF), jnp.float32)] * 2
            + [pltpu.VMEM((FCH, HALF), jnp.float32)] * 2
            + [
                pltpu.VMEM((L,), jnp.float32),
                pltpu.VMEM_SHARED((N_NODES, 2, HALF), jnp.float32),
                pltpu.SemaphoreType.DMA,
                pltpu.SemaphoreType.DMA,
                pltpu.SemaphoreType.DMA,
                pltpu.SemaphoreType.DMA,
                pltpu.SemaphoreType.DMA,
            ]
        ),
    )(_body)
    return f(x, idx, bvec)
